# baseline (speedup 1.0000x reference)
import jax
import jax.numpy as jnp
from jax import lax
from jax.experimental import pallas as pl
from jax.experimental.pallas import tpu as pltpu
from jax.experimental.pallas import tpu_sc as plsc

_HIST = 4096
_D = 16384
_B = 64
_NC = 2
_NS = 16
_NW = _NC * _NS
_RPW = _B // _NW
_L = 16


def _sc_body(fake_hbm, hist_hbm, mask_hbm, idx_hbm, out_hbm,
             idx_v, mask_v, sem0, sem1):
    wid = lax.axis_index("s") * _NC + lax.axis_index("c")
    pltpu.sync_copy(idx_hbm, idx_v)
    pltpu.sync_copy(mask_hbm, mask_v)

    jv0 = lax.iota(jnp.int32, _L)
    neg1 = jnp.full((_L,), -1, jnp.int32)
    sems = (sem0, sem1)

    for r in range(_RPW):
        i = wid * _RPW + r
        idx_acc = neg1
        mask_acc = neg1
        for k in range(_B // _L):
            jv = jv0 + (k * _L)
            onehot = jv == i
            idx_blk = idx_v[pl.ds(k * _L, _L)]
            mask_blk = mask_v[pl.ds(k * _L, _L)]
            idx_acc = jnp.maximum(idx_acc, jnp.where(onehot, idx_blk, neg1))
            mask_acc = jnp.maximum(mask_acc, jnp.where(onehot, mask_blk, neg1))
        idx_i = jnp.max(idx_acc)
        mask_i = jnp.max(mask_acc)

        best_acc = neg1
        for k in range(_B // _L):
            jv = jv0 + (k * _L)
            idx_blk = idx_v[pl.ds(k * _L, _L)]
            mask_blk = mask_v[pl.ds(k * _L, _L)]
            hit = (idx_blk == idx_i) & (mask_blk == 1) & (jv < i)
            best_acc = jnp.maximum(best_acc, jnp.where(hit, jv, neg1))
        best = jnp.max(best_acc)

        use_hist = (mask_i == 1) & (best < 0)
        frow = jnp.where(mask_i == 1, jnp.maximum(best, 0), i)

        @pl.when(use_hist)
        def _():
            pltpu.async_copy(hist_hbm.at[idx_i], out_hbm.at[i], sems[r])

        @pl.when(jnp.logical_not(use_hist))
        def _():
            pltpu.async_copy(fake_hbm.at[frow], out_hbm.at[i], sems[r])

    for r in range(_RPW):
        i = wid * _RPW + r
        pltpu.make_async_copy(fake_hbm.at[0], out_hbm.at[i], sems[r]).wait()


def kernel(fake, history, swap_mask, swap_idx):
    mesh = plsc.VectorSubcoreMesh(core_axis_name="c", subcore_axis_name="s")
    f = pl.kernel(
        _sc_body,
        out_type=jax.ShapeDtypeStruct((_B, _D), jnp.float32),
        mesh=mesh,
        compiler_params=pltpu.CompilerParams(needs_layout_passes=False),
        scratch_types=[
            pltpu.VMEM((_B,), jnp.int32),
            pltpu.VMEM((_B,), jnp.int32),
            pltpu.SemaphoreType.DMA,
            pltpu.SemaphoreType.DMA,
        ],
    )
    return f(fake, history, swap_mask, swap_idx)


# speedup vs baseline: 5.9029x; 5.9029x over previous
import jax
import jax.numpy as jnp
from jax import lax
from jax.experimental import pallas as pl
from jax.experimental.pallas import tpu as pltpu
from jax.experimental.pallas import tpu_sc as plsc

_HIST = 4096
_D = 16384
_B = 64
_NC = 2
_NS = 16
_NW = _NC * _NS
_RPW = _B // _NW
_L = 16


def _sc_body(fake_hbm, hist_hbm, mask_hbm, idx_hbm, out_hbm,
             idx_v, mask_v, buf_v, sem0, sem1, osem0, osem1):
    wid = lax.axis_index("s") * _NC + lax.axis_index("c")
    pltpu.sync_copy(idx_hbm, idx_v)
    pltpu.sync_copy(mask_hbm, mask_v)

    jv0 = lax.iota(jnp.int32, _L)
    neg1 = jnp.full((_L,), -1, jnp.int32)
    sems = (sem0, sem1)
    osems = (osem0, osem1)

    for r in range(_RPW):
        i = wid * _RPW + r
        idx_acc = neg1
        mask_acc = neg1
        for k in range(_B // _L):
            jv = jv0 + (k * _L)
            onehot = jv == i
            idx_blk = idx_v[pl.ds(k * _L, _L)]
            mask_blk = mask_v[pl.ds(k * _L, _L)]
            idx_acc = jnp.maximum(idx_acc, jnp.where(onehot, idx_blk, neg1))
            mask_acc = jnp.maximum(mask_acc, jnp.where(onehot, mask_blk, neg1))
        idx_i = jnp.max(idx_acc)
        mask_i = jnp.max(mask_acc)

        best_acc = neg1
        for k in range(_B // _L):
            jv = jv0 + (k * _L)
            idx_blk = idx_v[pl.ds(k * _L, _L)]
            mask_blk = mask_v[pl.ds(k * _L, _L)]
            hit = (idx_blk == idx_i) & (mask_blk == 1) & (jv < i)
            best_acc = jnp.maximum(best_acc, jnp.where(hit, jv, neg1))
        best = jnp.max(best_acc)

        use_hist = (mask_i == 1) & (best < 0)
        frow = jnp.where(mask_i == 1, jnp.maximum(best, 0), i)

        @pl.when(use_hist)
        def _():
            pltpu.async_copy(hist_hbm.at[idx_i], buf_v.at[r], sems[r])

        @pl.when(jnp.logical_not(use_hist))
        def _():
            pltpu.async_copy(fake_hbm.at[frow], buf_v.at[r], sems[r])

    for r in range(_RPW):
        i = wid * _RPW + r
        pltpu.make_async_copy(fake_hbm.at[0], buf_v.at[r], sems[r]).wait()
        pltpu.async_copy(buf_v.at[r], out_hbm.at[i], osems[r])

    for r in range(_RPW):
        i = wid * _RPW + r
        pltpu.make_async_copy(buf_v.at[r], out_hbm.at[i], osems[r]).wait()


def kernel(fake, history, swap_mask, swap_idx):
    mesh = plsc.VectorSubcoreMesh(core_axis_name="c", subcore_axis_name="s")
    f = pl.kernel(
        _sc_body,
        out_type=jax.ShapeDtypeStruct((_B, _D), jnp.float32),
        mesh=mesh,
        compiler_params=pltpu.CompilerParams(needs_layout_passes=False),
        scratch_types=[
            pltpu.VMEM((_B,), jnp.int32),
            pltpu.VMEM((_B,), jnp.int32),
            pltpu.VMEM((_RPW, _D), jnp.float32),
            pltpu.SemaphoreType.DMA,
            pltpu.SemaphoreType.DMA,
            pltpu.SemaphoreType.DMA,
            pltpu.SemaphoreType.DMA,
        ],
    )
    return f(fake, history, swap_mask, swap_idx)


# speedup vs baseline: 5.9222x; 1.0033x over previous
import jax
import jax.numpy as jnp
from jax import lax
from jax.experimental import pallas as pl
from jax.experimental.pallas import tpu as pltpu
from jax.experimental.pallas import tpu_sc as plsc

_HIST = 4096
_D = 16384
_B = 64
_NC = 2
_NS = 16
_NW = _NC * _NS
_RPW = _B // _NW
_L = 16


def _sc_body(fake_hbm, hist_hbm, mask_hbm, idx_hbm, out_hbm,
             idx_v, mask_v, buf_v, sem0, sem1, osem0, osem1):
    wid = lax.axis_index("s") * _NC + lax.axis_index("c")
    pltpu.sync_copy(idx_hbm, idx_v)
    pltpu.sync_copy(mask_hbm, mask_v)

    jv0 = lax.iota(jnp.int32, _L)
    neg1 = jnp.full((_L,), -1, jnp.int32)
    sems = (sem0, sem1)
    osems = (osem0, osem1)

    for r in range(_RPW):
        i = wid * _RPW + r

        def _extract(k, acc):
            jv = jv0 + k * _L
            onehot = jv == i
            idx_blk = idx_v[pl.ds(k * _L, _L)]
            mask_blk = mask_v[pl.ds(k * _L, _L)]
            return (jnp.maximum(acc[0], jnp.where(onehot, idx_blk, neg1)),
                    jnp.maximum(acc[1], jnp.where(onehot, mask_blk, neg1)))

        idx_acc, mask_acc = lax.fori_loop(0, _B // _L, _extract, (neg1, neg1))
        idx_i = jnp.max(idx_acc)
        mask_i = jnp.max(mask_acc)

        def _chain(k, acc):
            jv = jv0 + k * _L
            idx_blk = idx_v[pl.ds(k * _L, _L)]
            mask_blk = mask_v[pl.ds(k * _L, _L)]
            hit = (idx_blk == idx_i) & (mask_blk == 1) & (jv < i)
            return jnp.maximum(acc, jnp.where(hit, jv, neg1))

        best = jnp.max(lax.fori_loop(0, _B // _L, _chain, neg1))

        use_hist = (mask_i == 1) & (best < 0)
        frow = jnp.where(mask_i == 1, jnp.maximum(best, 0), i)

        @pl.when(use_hist)
        def _():
            pltpu.async_copy(hist_hbm.at[idx_i], buf_v.at[r], sems[r])

        @pl.when(jnp.logical_not(use_hist))
        def _():
            pltpu.async_copy(fake_hbm.at[frow], buf_v.at[r], sems[r])

    for r in range(_RPW):
        i = wid * _RPW + r
        pltpu.make_async_copy(fake_hbm.at[0], buf_v.at[r], sems[r]).wait()
        pltpu.async_copy(buf_v.at[r], out_hbm.at[i], osems[r])

    for r in range(_RPW):
        i = wid * _RPW + r
        pltpu.make_async_copy(buf_v.at[r], out_hbm.at[i], osems[r]).wait()


def kernel(fake, history, swap_mask, swap_idx):
    mesh = plsc.VectorSubcoreMesh(core_axis_name="c", subcore_axis_name="s")
    f = pl.kernel(
        _sc_body,
        out_type=jax.ShapeDtypeStruct((_B, _D), jnp.float32),
        mesh=mesh,
        compiler_params=pltpu.CompilerParams(needs_layout_passes=False),
        scratch_types=[
            pltpu.VMEM((_B,), jnp.int32),
            pltpu.VMEM((_B,), jnp.int32),
            pltpu.VMEM((_RPW, _D), jnp.float32),
            pltpu.SemaphoreType.DMA,
            pltpu.SemaphoreType.DMA,
            pltpu.SemaphoreType.DMA,
            pltpu.SemaphoreType.DMA,
        ],
    )
    return f(fake, history, swap_mask, swap_idx)


# speedup vs baseline: 6.0384x; 1.0196x over previous
import jax
import jax.numpy as jnp
from jax import lax
from jax.experimental import pallas as pl
from jax.experimental.pallas import tpu as pltpu
from jax.experimental.pallas import tpu_sc as plsc

_HIST = 4096
_D = 16384
_B = 64
_NC = 2
_NS = 16
_NW = _NC * _NS
_RPW = _B // _NW
_L = 16


def _sc_body(fake_hbm, hist_hbm, mask_hbm, idx_hbm, out_hbm,
             idx_v, mask_v, buf_v, sem0, sem1, osem0, osem1, isem):
    wid = lax.axis_index("s") * _NC + lax.axis_index("c")
    stage_i = pltpu.async_copy(idx_hbm, idx_v, isem)
    stage_m = pltpu.async_copy(mask_hbm, mask_v, isem)
    stage_i.wait()
    stage_m.wait()

    jv0 = lax.iota(jnp.int32, _L)
    neg1 = jnp.full((_L,), -1, jnp.int32)
    sems = (sem0, sem1)
    osems = (osem0, osem1)

    for r in range(_RPW):
        i = wid * _RPW + r

        bi = jnp.full((_L,), i, jnp.int32)
        gidx = plsc.load_gather(idx_v, [bi])
        gmask = plsc.load_gather(mask_v, [bi])
        idx_i = jnp.max(gidx)
        mask_i = jnp.max(gmask)

        def _chain(k, acc):
            jv = jv0 + k * _L
            idx_blk = idx_v[pl.ds(k * _L, _L)]
            mask_blk = mask_v[pl.ds(k * _L, _L)]
            hit = (idx_blk == gidx) & (mask_blk == 1) & (jv < i)
            return jnp.maximum(acc, jnp.where(hit, jv, neg1))

        best = jnp.max(lax.fori_loop(0, _B // _L, _chain, neg1))

        use_hist = (mask_i == 1) & (best < 0)
        frow = jnp.where(mask_i == 1, jnp.maximum(best, 0), i)

        @pl.when(use_hist)
        def _():
            pltpu.async_copy(hist_hbm.at[idx_i], buf_v.at[r], sems[r])

        @pl.when(jnp.logical_not(use_hist))
        def _():
            pltpu.async_copy(fake_hbm.at[frow], buf_v.at[r], sems[r])

    for r in range(_RPW):
        i = wid * _RPW + r
        pltpu.make_async_copy(fake_hbm.at[0], buf_v.at[r], sems[r]).wait()
        pltpu.async_copy(buf_v.at[r], out_hbm.at[i], osems[r])

    for r in range(_RPW):
        i = wid * _RPW + r
        pltpu.make_async_copy(buf_v.at[r], out_hbm.at[i], osems[r]).wait()


def kernel(fake, history, swap_mask, swap_idx):
    mesh = plsc.VectorSubcoreMesh(core_axis_name="c", subcore_axis_name="s")
    f = pl.kernel(
        _sc_body,
        out_type=jax.ShapeDtypeStruct((_B, _D), jnp.float32),
        mesh=mesh,
        compiler_params=pltpu.CompilerParams(needs_layout_passes=False),
        scratch_types=[
            pltpu.VMEM((_B,), jnp.int32),
            pltpu.VMEM((_B,), jnp.int32),
            pltpu.VMEM((_RPW, _D), jnp.float32),
            pltpu.SemaphoreType.DMA,
            pltpu.SemaphoreType.DMA,
            pltpu.SemaphoreType.DMA,
            pltpu.SemaphoreType.DMA,
            pltpu.SemaphoreType.DMA,
        ],
    )
    return f(fake, history, swap_mask, swap_idx)


# speedup vs baseline: 6.0820x; 1.0072x over previous
import jax
import jax.numpy as jnp
from jax import lax
from jax.experimental import pallas as pl
from jax.experimental.pallas import tpu as pltpu
from jax.experimental.pallas import tpu_sc as plsc

_HIST = 4096
_D = 16384
_B = 64
_NC = 2
_NS = 16
_NW = _NC * _NS
_RPW = _B // _NW
_L = 16
_NCH = 4
_CH = _D // _NCH


def _sc_body(fake_hbm, hist_hbm, mask_hbm, idx_hbm, out_hbm,
             idx_v, mask_v, buf_v, sem0, sem1, osem0, osem1, isem):
    wid = lax.axis_index("s") * _NC + lax.axis_index("c")
    stage_i = pltpu.async_copy(idx_hbm, idx_v, isem)
    stage_m = pltpu.async_copy(mask_hbm, mask_v, isem)
    stage_i.wait()
    stage_m.wait()

    jv0 = lax.iota(jnp.int32, _L)
    neg1 = jnp.full((_L,), -1, jnp.int32)
    sems = (sem0, sem1)
    osems = (osem0, osem1)

    for r in range(_RPW):
        i = wid * _RPW + r

        bi = jnp.full((_L,), i, jnp.int32)
        gidx = plsc.load_gather(idx_v, [bi])
        gmask = plsc.load_gather(mask_v, [bi])
        idx_i = jnp.max(gidx)
        mask_i = jnp.max(gmask)

        def _chain(k, acc):
            jv = jv0 + k * _L
            idx_blk = idx_v[pl.ds(k * _L, _L)]
            mask_blk = mask_v[pl.ds(k * _L, _L)]
            hit = (idx_blk == gidx) & (mask_blk == 1) & (jv < i)
            return jnp.maximum(acc, jnp.where(hit, jv, neg1))

        best = jnp.max(lax.fori_loop(0, _B // _L, _chain, neg1))

        use_hist = (mask_i == 1) & (best < 0)
        frow = jnp.where(mask_i == 1, jnp.maximum(best, 0), i)

        @pl.when(use_hist)
        def _():
            for c in range(_NCH):
                pltpu.async_copy(hist_hbm.at[idx_i, pl.ds(c * _CH, _CH)],
                                 buf_v.at[r, pl.ds(c * _CH, _CH)], sems[r])

        @pl.when(jnp.logical_not(use_hist))
        def _():
            for c in range(_NCH):
                pltpu.async_copy(fake_hbm.at[frow, pl.ds(c * _CH, _CH)],
                                 buf_v.at[r, pl.ds(c * _CH, _CH)], sems[r])

    for r in range(_RPW):
        i = wid * _RPW + r
        for c in range(_NCH):
            pltpu.make_async_copy(fake_hbm.at[0, pl.ds(0, _CH)],
                                  buf_v.at[r, pl.ds(c * _CH, _CH)],
                                  sems[r]).wait()
            pltpu.async_copy(buf_v.at[r, pl.ds(c * _CH, _CH)],
                             out_hbm.at[i, pl.ds(c * _CH, _CH)], osems[r])

    for r in range(_RPW):
        i = wid * _RPW + r
        for c in range(_NCH):
            pltpu.make_async_copy(buf_v.at[r, pl.ds(c * _CH, _CH)],
                                  out_hbm.at[i, pl.ds(c * _CH, _CH)],
                                  osems[r]).wait()


def kernel(fake, history, swap_mask, swap_idx):
    mesh = plsc.VectorSubcoreMesh(core_axis_name="c", subcore_axis_name="s")
    f = pl.kernel(
        _sc_body,
        out_type=jax.ShapeDtypeStruct((_B, _D), jnp.float32),
        mesh=mesh,
        compiler_params=pltpu.CompilerParams(needs_layout_passes=False),
        scratch_types=[
            pltpu.VMEM((_B,), jnp.int32),
            pltpu.VMEM((_B,), jnp.int32),
            pltpu.VMEM((_RPW, _D), jnp.float32),
            pltpu.SemaphoreType.DMA,
            pltpu.SemaphoreType.DMA,
            pltpu.SemaphoreType.DMA,
            pltpu.SemaphoreType.DMA,
            pltpu.SemaphoreType.DMA,
        ],
    )
    return f(fake, history, swap_mask, swap_idx)
